# baseline (speedup 1.0000x reference)
import functools

import jax
import jax.numpy as jnp
from jax import lax
from jax.experimental import pallas as pl
from jax.experimental.pallas import tpu as pltpu
from jax.experimental.pallas import tpu_sc as plsc

N = 10000
E = 320000
D = 128
G = 64
NC, NS = 2, 16
NW = NC * NS
EPT = E // NW
CHUNK = 80
NCHUNK = EPT // CHUNK
RPT = N // NS

_sc_mesh = plsc.VectorSubcoreMesh(core_axis_name="c", subcore_axis_name="s")


@functools.partial(
    pl.kernel,
    out_type=jax.ShapeDtypeStruct((NC, NS, RPT, 16), jnp.float32),
    mesh=_sc_mesh,
    scratch_types=[
        pltpu.VMEM((NCHUNK, CHUNK), jnp.int32),
        pltpu.VMEM((CHUNK, 16), jnp.float32),
        pltpu.VMEM((CHUNK, 16), jnp.float32),
        pltpu.VMEM_SHARED((N, 16), jnp.float32),
    ],
)
def _deg_kernel(dst_hbm, out_hbm, didx, ones_v, zbuf, acc):
    c = lax.axis_index("c")
    s = lax.axis_index("s")
    wid = c * NS + s

    ones16 = jnp.full((16,), 1.0, jnp.float32)
    zeros16 = jnp.zeros((16,), jnp.float32)

    def fill_ones(i, carry):
        ones_v[i, :] = ones16
        return carry

    lax.fori_loop(0, CHUNK, fill_ones, 0)

    def fill_zero(i, carry):
        zbuf[i, :] = zeros16
        return carry

    lax.fori_loop(0, CHUNK, fill_zero, 0)

    for t in range(7):
        pltpu.sync_copy(zbuf, acc.at[pl.ds(s * RPT + t * CHUNK, CHUNK)])
    pltpu.sync_copy(zbuf.at[pl.ds(0, RPT - 7 * CHUNK)],
                    acc.at[pl.ds(s * RPT + 7 * CHUNK, RPT - 7 * CHUNK)])
    plsc.subcore_barrier()

    pltpu.sync_copy(dst_hbm.at[wid], didx)

    def body(j, carry):
        pltpu.sync_copy(ones_v, acc.at[didx.at[j]], add=True)
        return carry

    lax.fori_loop(0, NCHUNK, body, 0)
    plsc.subcore_barrier()

    pltpu.sync_copy(acc.at[pl.ds(s * RPT, RPT)], out_hbm.at[c, s])


@functools.partial(
    pl.kernel,
    out_type=jax.ShapeDtypeStruct((NC, NS, RPT, D), jnp.float32),
    mesh=_sc_mesh,
    scratch_types=[
        pltpu.VMEM((EPT,), jnp.int32),
        pltpu.VMEM((NCHUNK, CHUNK), jnp.int32),
        pltpu.VMEM((CHUNK, D), jnp.float32),
        pltpu.VMEM((CHUNK, D), jnp.float32),
        pltpu.VMEM_SHARED((N, D), jnp.float32),
        pltpu.SemaphoreType.DMA,
        pltpu.SemaphoreType.DMA,
    ],
)
def _prop_kernel(q_hbm, src_flat_hbm, dst_hbm, out_hbm,
                 sidx, didx, rows_a, rows_b, acc, sem_a, sem_b):
    c = lax.axis_index("c")
    s = lax.axis_index("s")
    wid = c * NS + s

    zeros16 = jnp.zeros((16,), jnp.float32)

    def fill_zero(i, carry):
        for k in range(D // 16):
            rows_a[i, pl.ds(k * 16, 16)] = zeros16
        return carry

    lax.fori_loop(0, CHUNK, fill_zero, 0)
    for t in range(7):
        pltpu.sync_copy(rows_a, acc.at[pl.ds(s * RPT + t * CHUNK, CHUNK)])
    pltpu.sync_copy(rows_a.at[pl.ds(0, RPT - 7 * CHUNK)],
                    acc.at[pl.ds(s * RPT + 7 * CHUNK, RPT - 7 * CHUNK)])
    plsc.subcore_barrier()

    pltpu.sync_copy(src_flat_hbm.at[pl.ds(wid * EPT, EPT)], sidx)
    pltpu.sync_copy(dst_hbm.at[wid], didx)

    pltpu.async_copy(q_hbm.at[sidx.at[pl.ds(0, CHUNK)]], rows_a, sem_a)

    def body(j, carry):
        @pl.when(j % 2 == 0)
        def _even():
            pltpu.make_async_copy(
                q_hbm.at[sidx.at[pl.ds(j * CHUNK, CHUNK)]], rows_a,
                sem_a).wait()

            @pl.when(j + 1 < NCHUNK)
            def _pf():
                pltpu.async_copy(
                    q_hbm.at[sidx.at[pl.ds((j + 1) * CHUNK, CHUNK)]], rows_b,
                    sem_b)

            pltpu.sync_copy(rows_a, acc.at[didx.at[j]], add=True)

        @pl.when(j % 2 == 1)
        def _odd():
            pltpu.make_async_copy(
                q_hbm.at[sidx.at[pl.ds(j * CHUNK, CHUNK)]], rows_b,
                sem_b).wait()

            @pl.when(j + 1 < NCHUNK)
            def _pf():
                pltpu.async_copy(
                    q_hbm.at[sidx.at[pl.ds((j + 1) * CHUNK, CHUNK)]], rows_a,
                    sem_a)

            pltpu.sync_copy(rows_b, acc.at[didx.at[j]], add=True)

        return carry

    lax.fori_loop(0, NCHUNK, body, 0)
    plsc.subcore_barrier()

    pltpu.sync_copy(acc.at[pl.ds(s * RPT, RPT)], out_hbm.at[c, s])


NB = 2000
NBLK = N // NB


def _dis_blk(d0_ref, d1_ref):
    deg = d0_ref[:, 0:1] + d1_ref[:, 0:1] + 1.0
    return lax.rsqrt(deg)


def _tc_first_body(d0_ref, d1_ref, x_ref, w_ref, q_ref):
    dis = _dis_blk(d0_ref, d1_ref)
    q_ref[...] = dis * jnp.dot(x_ref[...], w_ref[...],
                               preferred_element_type=jnp.float32)


def _tc_mid_body(d0_ref, d1_ref, p0_ref, p1_ref, q_ref, b_ref, w_ref, qo_ref):
    dis = _dis_blk(d0_ref, d1_ref)
    h = dis * (p0_ref[...] + p1_ref[...] + q_ref[...]) + b_ref[...]
    h = jnp.maximum(h, 0.0)
    qo_ref[...] = dis * jnp.dot(h, w_ref[...],
                                preferred_element_type=jnp.float32)


def _tc_final_body(d0_ref, d1_ref, p0_ref, p1_ref, q_ref, b_ref, batch_ref,
                   wl_ref, bl_ref, out_ref, sums_ref, cnts_ref):
    i = pl.program_id(0)

    @pl.when(i == 0)
    def _init():
        sums_ref[...] = jnp.zeros((G, D), jnp.float32)
        cnts_ref[...] = jnp.zeros((G, 128), jnp.float32)

    dis = _dis_blk(d0_ref, d1_ref)
    h = dis * (p0_ref[...] + p1_ref[...] + q_ref[...]) + b_ref[...]
    gid = lax.broadcasted_iota(jnp.int32, (G, NB), 0)
    onehot = (gid == batch_ref[0]).astype(jnp.float32)
    sums_ref[...] += jnp.dot(onehot, h, preferred_element_type=jnp.float32)
    cnts_ref[:, 0:1] += jnp.sum(onehot, axis=1, keepdims=True)

    @pl.when(i == NBLK - 1)
    def _fin():
        g = sums_ref[...] / jnp.maximum(cnts_ref[:, 0:1], 1.0)
        logits = jnp.dot(g, wl_ref[...], preferred_element_type=jnp.float32)
        logits = logits + bl_ref[...]
        m = jnp.max(logits, axis=1, keepdims=True)
        lse = jnp.log(jnp.sum(jnp.exp(logits - m), axis=1, keepdims=True)) + m
        out_ref[...] = logits - lse


_row_blk = pl.BlockSpec((NB, 16), lambda i: (i, 0))
_feat_blk = pl.BlockSpec((NB, D), lambda i: (i, 0))
_whole_w = pl.BlockSpec((D, D), lambda i: (0, 0))
_whole_b = pl.BlockSpec((1, D), lambda i: (0, 0))

_tc_first = pl.pallas_call(
    _tc_first_body,
    grid=(NBLK,),
    in_specs=[_row_blk, _row_blk, _feat_blk, _whole_w],
    out_specs=_feat_blk,
    out_shape=jax.ShapeDtypeStruct((N, D), jnp.float32))

_tc_mid = pl.pallas_call(
    _tc_mid_body,
    grid=(NBLK,),
    in_specs=[_row_blk, _row_blk, _feat_blk, _feat_blk, _feat_blk, _whole_b,
              _whole_w],
    out_specs=_feat_blk,
    out_shape=jax.ShapeDtypeStruct((N, D), jnp.float32))

_tc_final = pl.pallas_call(
    _tc_final_body,
    grid=(NBLK,),
    in_specs=[_row_blk, _row_blk, _feat_blk, _feat_blk, _feat_blk, _whole_b,
              pl.BlockSpec((1, 1, NB), lambda i: (i, 0, 0)),
              pl.BlockSpec((D, 4), lambda i: (0, 0)),
              pl.BlockSpec((1, 4), lambda i: (0, 0))],
    out_specs=pl.BlockSpec((G, 4), lambda i: (0, 0)),
    out_shape=jax.ShapeDtypeStruct((G, 4), jnp.float32),
    scratch_shapes=[pltpu.VMEM((G, D), jnp.float32),
                    pltpu.VMEM((G, 128), jnp.float32)])


def kernel(x, edge_index, batch, W1, b1, W2, b2, W3, b3, Wl, bl):
    src_flat = edge_index[0].astype(jnp.int32)
    dst = edge_index[1].astype(jnp.int32).reshape(NW, NCHUNK, CHUNK)
    batch3d = batch.astype(jnp.int32).reshape(NBLK, 1, NB)

    degp = _deg_kernel(dst)
    d0 = degp[0].reshape(N, 16)
    d1 = degp[1].reshape(N, 16)
    q1 = _tc_first(d0, d1, x, W1)
    p = _prop_kernel(q1, src_flat, dst)
    p0, p1 = p[0].reshape(N, D), p[1].reshape(N, D)
    q2 = _tc_mid(d0, d1, p0, p1, q1, b1.reshape(1, D), W2)
    p = _prop_kernel(q2, src_flat, dst)
    p0, p1 = p[0].reshape(N, D), p[1].reshape(N, D)
    q3 = _tc_mid(d0, d1, p0, p1, q2, b2.reshape(1, D), W3)
    p = _prop_kernel(q3, src_flat, dst)
    p0, p1 = p[0].reshape(N, D), p[1].reshape(N, D)
    return _tc_final(d0, d1, p0, p1, q3, b3.reshape(1, D), batch3d, Wl,
                     bl.reshape(1, 4))


# speedup vs baseline: 1.0022x; 1.0022x over previous
import functools

import jax
import jax.numpy as jnp
from jax import lax
from jax.experimental import pallas as pl
from jax.experimental.pallas import tpu as pltpu
from jax.experimental.pallas import tpu_sc as plsc

N = 10000
E = 320000
D = 128
G = 64
NC, NS = 2, 16
NW = NC * NS
EPT = E // NW
CHUNK = 80
NCHUNK = EPT // CHUNK
RPT = N // NS

_sc_mesh = plsc.VectorSubcoreMesh(core_axis_name="c", subcore_axis_name="s")


@functools.partial(
    pl.kernel,
    out_type=jax.ShapeDtypeStruct((NC, NS, RPT, 16), jnp.float32),
    mesh=_sc_mesh,
    scratch_types=[
        pltpu.VMEM((NCHUNK, CHUNK), jnp.int32),
        pltpu.VMEM((CHUNK, 16), jnp.float32),
        pltpu.VMEM((CHUNK, 16), jnp.float32),
        pltpu.VMEM_SHARED((N, 16), jnp.float32),
    ],
)
def _deg_kernel(dst_hbm, out_hbm, didx, ones_v, zbuf, acc):
    c = lax.axis_index("c")
    s = lax.axis_index("s")
    wid = c * NS + s

    ones16 = jnp.full((16,), 1.0, jnp.float32)
    zeros16 = jnp.zeros((16,), jnp.float32)

    def fill_ones(i, carry):
        ones_v[i, :] = ones16
        return carry

    lax.fori_loop(0, CHUNK, fill_ones, 0)

    def fill_zero(i, carry):
        zbuf[i, :] = zeros16
        return carry

    lax.fori_loop(0, CHUNK, fill_zero, 0)

    for t in range(7):
        pltpu.sync_copy(zbuf, acc.at[pl.ds(s * RPT + t * CHUNK, CHUNK)])
    pltpu.sync_copy(zbuf.at[pl.ds(0, RPT - 7 * CHUNK)],
                    acc.at[pl.ds(s * RPT + 7 * CHUNK, RPT - 7 * CHUNK)])
    plsc.subcore_barrier()

    pltpu.sync_copy(dst_hbm.at[wid], didx)

    def body(j, carry):
        pltpu.sync_copy(ones_v, acc.at[didx.at[j]], add=True)
        return carry

    lax.fori_loop(0, NCHUNK, body, 0)
    plsc.subcore_barrier()

    pltpu.sync_copy(acc.at[pl.ds(s * RPT, RPT)], out_hbm.at[c, s])


@functools.partial(
    pl.kernel,
    out_type=jax.ShapeDtypeStruct((NC, NS, RPT, D), jnp.float32),
    mesh=_sc_mesh,
    scratch_types=[
        pltpu.VMEM((CHUNK, D), jnp.float32),
        pltpu.VMEM((CHUNK, D), jnp.float32),
        pltpu.VMEM((CHUNK, D), jnp.float32),
        pltpu.VMEM((CHUNK,), jnp.int32),
        pltpu.VMEM((CHUNK,), jnp.int32),
        pltpu.VMEM((CHUNK,), jnp.int32),
        pltpu.VMEM((CHUNK,), jnp.int32),
        pltpu.VMEM((CHUNK,), jnp.int32),
        pltpu.VMEM((CHUNK,), jnp.int32),
        pltpu.VMEM_SHARED((N, D), jnp.float32),
        pltpu.SemaphoreType.DMA,
        pltpu.SemaphoreType.DMA,
        pltpu.SemaphoreType.DMA,
    ],
)
def _prop_kernel(q_hbm, src_hbm, dst_hbm, out_hbm,
                 rows0, rows1, rows2, si0, si1, si2, di0, di1, di2,
                 acc, sem_i, sem_g, sem_s):
    c = lax.axis_index("c")
    s = lax.axis_index("s")
    wid = c * NS + s
    ebase = wid * EPT
    rows = (rows0, rows1, rows2)
    si = (si0, si1, si2)
    di = (di0, di1, di2)

    zeros16 = jnp.zeros((16,), jnp.float32)

    def fire_idx(j, k):
        pltpu.async_copy(src_hbm.at[pl.ds(ebase + j * CHUNK, CHUNK)], si[k],
                         sem_i)
        pltpu.async_copy(dst_hbm.at[pl.ds(ebase + j * CHUNK, CHUNK)], di[k],
                         sem_i)

    def wait_idx(j, k):
        pltpu.make_async_copy(src_hbm.at[pl.ds(ebase + j * CHUNK, CHUNK)],
                              si[k], sem_i).wait()
        pltpu.make_async_copy(dst_hbm.at[pl.ds(ebase + j * CHUNK, CHUNK)],
                              di[k], sem_i).wait()

    fire_idx(0, 0)
    fire_idx(1, 1)

    def fill_zero(i, carry):
        for k in range(D // 16):
            rows0[i, pl.ds(k * 16, 16)] = zeros16
        return carry

    lax.fori_loop(0, CHUNK, fill_zero, 0)
    for t in range(7):
        pltpu.sync_copy(rows0, acc.at[pl.ds(s * RPT + t * CHUNK, CHUNK)])
    pltpu.sync_copy(rows0.at[pl.ds(0, RPT - 7 * CHUNK)],
                    acc.at[pl.ds(s * RPT + 7 * CHUNK, RPT - 7 * CHUNK)])
    plsc.subcore_barrier()

    wait_idx(0, 0)
    pltpu.async_copy(q_hbm.at[si0], rows0, sem_g)

    def body(j, carry):
        for k in range(3):
            k1 = (k + 1) % 3
            k2 = (k + 2) % 3

            @pl.when(j % 3 == k)
            def _(k=k, k1=k1, k2=k2):
                pltpu.make_async_copy(q_hbm.at[si[k]], rows[k], sem_g).wait()
                pltpu.async_copy(rows[k], acc.at[di[k]], sem_s, add=True)

                @pl.when(j + 1 < NCHUNK)
                def _(k1=k1):
                    wait_idx(j + 1, k1)

                @pl.when(j >= 1)
                def _(k2=k2):
                    pltpu.make_async_copy(rows[k2], acc.at[di[k2]],
                                          sem_s).wait()

                @pl.when(j + 1 < NCHUNK)
                def _(k1=k1):
                    pltpu.async_copy(q_hbm.at[si[k1]], rows[k1], sem_g)

                @pl.when(j + 2 < NCHUNK)
                def _(k2=k2):
                    fire_idx(j + 2, k2)

        return carry

    lax.fori_loop(0, NCHUNK, body, 0)
    klast = (NCHUNK - 1) % 3
    pltpu.make_async_copy(rows[klast], acc.at[di[klast]], sem_s).wait()
    plsc.subcore_barrier()

    pltpu.sync_copy(acc.at[pl.ds(s * RPT, RPT)], out_hbm.at[c, s])


def _dis(d0_ref, d1_ref):
    deg = d0_ref[:, 0:1] + d1_ref[:, 0:1] + 1.0
    return lax.rsqrt(deg)


def _tc_first_body(d0_ref, d1_ref, x_ref, w_ref, q_ref):
    dis = _dis(d0_ref, d1_ref)
    q_ref[...] = dis * jnp.dot(x_ref[...], w_ref[...],
                               preferred_element_type=jnp.float32)


def _tc_mid_body(d0_ref, d1_ref, p0_ref, p1_ref, q_ref, b_ref, w_ref, qo_ref):
    dis = _dis(d0_ref, d1_ref)
    h = dis * (p0_ref[...] + p1_ref[...] + q_ref[...]) + b_ref[...]
    h = jnp.maximum(h, 0.0)
    qo_ref[...] = dis * jnp.dot(h, w_ref[...],
                                preferred_element_type=jnp.float32)


def _tc_final_body(d0_ref, d1_ref, p0_ref, p1_ref, q_ref, b_ref, batch_ref,
                   wl_ref, bl_ref, out_ref):
    dis = _dis(d0_ref, d1_ref)
    h = dis * (p0_ref[...] + p1_ref[...] + q_ref[...]) + b_ref[...]
    gid = lax.broadcasted_iota(jnp.int32, (G, N), 0)
    onehot = (gid == batch_ref[...]).astype(jnp.float32)
    sums = jnp.dot(onehot, h, preferred_element_type=jnp.float32)
    cnts = jnp.sum(onehot, axis=1, keepdims=True)
    g = sums / jnp.maximum(cnts, 1.0)
    logits = jnp.dot(g, wl_ref[...], preferred_element_type=jnp.float32)
    logits = logits + bl_ref[...]
    m = jnp.max(logits, axis=1, keepdims=True)
    lse = jnp.log(jnp.sum(jnp.exp(logits - m), axis=1, keepdims=True)) + m
    out_ref[...] = logits - lse


_tc_first = pl.pallas_call(
    _tc_first_body, out_shape=jax.ShapeDtypeStruct((N, D), jnp.float32))
_tc_mid = pl.pallas_call(
    _tc_mid_body, out_shape=jax.ShapeDtypeStruct((N, D), jnp.float32))
_tc_final = pl.pallas_call(
    _tc_final_body, out_shape=jax.ShapeDtypeStruct((G, 4), jnp.float32))


def kernel(x, edge_index, batch, W1, b1, W2, b2, W3, b3, Wl, bl):
    src_flat = edge_index[0].astype(jnp.int32)
    dst_flat = edge_index[1].astype(jnp.int32)
    dst3 = dst_flat.reshape(NW, NCHUNK, CHUNK)
    batch2d = batch.astype(jnp.int32).reshape(1, N)

    degp = _deg_kernel(dst3)
    d0 = degp[0].reshape(N, 16)
    d1 = degp[1].reshape(N, 16)
    q1 = _tc_first(d0, d1, x, W1)
    p = _prop_kernel(q1, src_flat, dst_flat)
    p0, p1 = p[0].reshape(N, D), p[1].reshape(N, D)
    q2 = _tc_mid(d0, d1, p0, p1, q1, b1.reshape(1, D), W2)
    p = _prop_kernel(q2, src_flat, dst_flat)
    p0, p1 = p[0].reshape(N, D), p[1].reshape(N, D)
    q3 = _tc_mid(d0, d1, p0, p1, q2, b2.reshape(1, D), W3)
    p = _prop_kernel(q3, src_flat, dst_flat)
    p0, p1 = p[0].reshape(N, D), p[1].reshape(N, D)
    return _tc_final(d0, d1, p0, p1, q3, b3.reshape(1, D), batch2d, Wl,
                     bl.reshape(1, 4))


# speedup vs baseline: 1.0853x; 1.0830x over previous
import functools

import jax
import jax.numpy as jnp
from jax import lax
from jax.experimental import pallas as pl
from jax.experimental.pallas import tpu as pltpu
from jax.experimental.pallas import tpu_sc as plsc

N = 10000
E = 320000
D = 128
G = 64
NC, NS = 2, 16
NW = NC * NS
EPT = E // NW
CHUNK = 80
NCHUNK = EPT // CHUNK
RPT = N // NS

_sc_mesh = plsc.VectorSubcoreMesh(core_axis_name="c", subcore_axis_name="s")


@functools.partial(
    pl.kernel,
    out_type=jax.ShapeDtypeStruct((NC, NS, RPT, 16), jnp.float32),
    mesh=_sc_mesh,
    scratch_types=[
        pltpu.VMEM((NCHUNK, CHUNK), jnp.int32),
        pltpu.VMEM((CHUNK, 16), jnp.float32),
        pltpu.VMEM((CHUNK, 16), jnp.float32),
        pltpu.VMEM_SHARED((N, 16), jnp.float32),
    ],
)
def _deg_kernel(dst_hbm, out_hbm, didx, ones_v, zbuf, acc):
    c = lax.axis_index("c")
    s = lax.axis_index("s")
    wid = c * NS + s

    ones16 = jnp.full((16,), 1.0, jnp.float32)
    zeros16 = jnp.zeros((16,), jnp.float32)

    def fill_ones(i, carry):
        ones_v[i, :] = ones16
        return carry

    lax.fori_loop(0, CHUNK, fill_ones, 0)

    def fill_zero(i, carry):
        zbuf[i, :] = zeros16
        return carry

    lax.fori_loop(0, CHUNK, fill_zero, 0)

    for t in range(7):
        pltpu.sync_copy(zbuf, acc.at[pl.ds(s * RPT + t * CHUNK, CHUNK)])
    pltpu.sync_copy(zbuf.at[pl.ds(0, RPT - 7 * CHUNK)],
                    acc.at[pl.ds(s * RPT + 7 * CHUNK, RPT - 7 * CHUNK)])
    plsc.subcore_barrier()

    pltpu.sync_copy(dst_hbm.at[wid], didx)

    def body(j, carry):
        pltpu.sync_copy(ones_v, acc.at[didx.at[j]], add=True)
        return carry

    lax.fori_loop(0, NCHUNK, body, 0)
    plsc.subcore_barrier()

    pltpu.sync_copy(acc.at[pl.ds(s * RPT, RPT)], out_hbm.at[c, s])


CH = 104
NCH = 96
TAIL = EPT - NCH * CH


@functools.partial(
    pl.kernel,
    out_type=jax.ShapeDtypeStruct((NC, NS, RPT, D), jnp.float32),
    mesh=_sc_mesh,
    scratch_types=[
        pltpu.VMEM((EPT,), jnp.int32),
        pltpu.VMEM((NCH, CH), jnp.int32),
        pltpu.VMEM((TAIL,), jnp.int32),
        pltpu.VMEM((CH, D), jnp.float32),
        pltpu.VMEM((CH, D), jnp.float32),
        pltpu.VMEM_SHARED((N, D), jnp.float32),
        pltpu.SemaphoreType.DMA,
        pltpu.SemaphoreType.DMA,
    ],
)
def _prop_kernel(q_hbm, src_flat_hbm, dst_hbm, dst_flat_hbm, out_hbm,
                 sidx, didx, didx_t, rows_a, rows_b, acc, sem_a, sem_b):
    c = lax.axis_index("c")
    s = lax.axis_index("s")
    wid = c * NS + s

    zeros16 = jnp.zeros((16,), jnp.float32)

    def fill_zero(i, carry):
        for k in range(D // 16):
            rows_a[i, pl.ds(k * 16, 16)] = zeros16
        return carry

    lax.fori_loop(0, CH, fill_zero, 0)
    for t in range(6):
        pltpu.sync_copy(rows_a, acc.at[pl.ds(s * RPT + t * CH, CH)])
    pltpu.sync_copy(rows_a.at[pl.ds(0, RPT - 6 * CH)],
                    acc.at[pl.ds(s * RPT + 6 * CH, RPT - 6 * CH)])
    plsc.subcore_barrier()

    pltpu.sync_copy(src_flat_hbm.at[pl.ds(wid * EPT, EPT)], sidx)
    pltpu.sync_copy(dst_hbm.at[wid], didx)
    pltpu.sync_copy(dst_flat_hbm.at[pl.ds(wid * EPT + NCH * CH, TAIL)],
                    didx_t)

    pltpu.async_copy(q_hbm.at[sidx.at[pl.ds(NCH * CH, TAIL)]],
                     rows_a.at[pl.ds(0, TAIL)], sem_a)
    pltpu.make_async_copy(q_hbm.at[sidx.at[pl.ds(NCH * CH, TAIL)]],
                          rows_a.at[pl.ds(0, TAIL)], sem_a).wait()
    pltpu.sync_copy(rows_a.at[pl.ds(0, TAIL)], acc.at[didx_t], add=True)

    pltpu.async_copy(q_hbm.at[sidx.at[pl.ds(0, CH)]], rows_a, sem_a)

    def body(j, carry):
        @pl.when(j % 2 == 0)
        def _even():
            pltpu.make_async_copy(
                q_hbm.at[sidx.at[pl.ds(j * CH, CH)]], rows_a, sem_a).wait()

            @pl.when(j + 1 < NCH)
            def _pf():
                pltpu.async_copy(
                    q_hbm.at[sidx.at[pl.ds((j + 1) * CH, CH)]], rows_b, sem_b)

            pltpu.sync_copy(rows_a, acc.at[didx.at[j]], add=True)

        @pl.when(j % 2 == 1)
        def _odd():
            pltpu.make_async_copy(
                q_hbm.at[sidx.at[pl.ds(j * CH, CH)]], rows_b, sem_b).wait()

            @pl.when(j + 1 < NCH)
            def _pf():
                pltpu.async_copy(
                    q_hbm.at[sidx.at[pl.ds((j + 1) * CH, CH)]], rows_a, sem_a)

            pltpu.sync_copy(rows_b, acc.at[didx.at[j]], add=True)

        return carry

    lax.fori_loop(0, NCH, body, 0)
    plsc.subcore_barrier()

    pltpu.sync_copy(acc.at[pl.ds(s * RPT, RPT)], out_hbm.at[c, s])


def _dis(d0_ref, d1_ref):
    deg = d0_ref[:, 0:1] + d1_ref[:, 0:1] + 1.0
    return lax.rsqrt(deg)


def _tc_first_body(d0_ref, d1_ref, x_ref, w_ref, q_ref):
    dis = _dis(d0_ref, d1_ref)
    q_ref[...] = dis * jnp.dot(x_ref[...], w_ref[...],
                               preferred_element_type=jnp.float32)


def _tc_mid_body(d0_ref, d1_ref, p0_ref, p1_ref, q_ref, b_ref, w_ref, qo_ref):
    dis = _dis(d0_ref, d1_ref)
    h = dis * (p0_ref[...] + p1_ref[...] + q_ref[...]) + b_ref[...]
    h = jnp.maximum(h, 0.0)
    qo_ref[...] = dis * jnp.dot(h, w_ref[...],
                                preferred_element_type=jnp.float32)


def _tc_final_body(d0_ref, d1_ref, p0_ref, p1_ref, q_ref, b_ref, batch_ref,
                   wl_ref, bl_ref, out_ref):
    dis = _dis(d0_ref, d1_ref)
    h = dis * (p0_ref[...] + p1_ref[...] + q_ref[...]) + b_ref[...]
    gid = lax.broadcasted_iota(jnp.int32, (G, N), 0)
    onehot = (gid == batch_ref[...]).astype(jnp.float32)
    sums = jnp.dot(onehot, h, preferred_element_type=jnp.float32)
    cnts = jnp.sum(onehot, axis=1, keepdims=True)
    g = sums / jnp.maximum(cnts, 1.0)
    logits = jnp.dot(g, wl_ref[...], preferred_element_type=jnp.float32)
    logits = logits + bl_ref[...]
    m = jnp.max(logits, axis=1, keepdims=True)
    lse = jnp.log(jnp.sum(jnp.exp(logits - m), axis=1, keepdims=True)) + m
    out_ref[...] = logits - lse


_tc_first = pl.pallas_call(
    _tc_first_body, out_shape=jax.ShapeDtypeStruct((N, D), jnp.float32))
_tc_mid = pl.pallas_call(
    _tc_mid_body, out_shape=jax.ShapeDtypeStruct((N, D), jnp.float32))
_tc_final = pl.pallas_call(
    _tc_final_body, out_shape=jax.ShapeDtypeStruct((G, 4), jnp.float32))


def kernel(x, edge_index, batch, W1, b1, W2, b2, W3, b3, Wl, bl):
    src_flat = edge_index[0].astype(jnp.int32)
    dst_flat = edge_index[1].astype(jnp.int32)
    dst3 = dst_flat.reshape(NW, NCHUNK, CHUNK)
    batch2d = batch.astype(jnp.int32).reshape(1, N)

    dstp3 = dst_flat.reshape(NW, EPT)[:, :NCH * CH].reshape(NW, NCH, CH)

    degp = _deg_kernel(dst3)
    d0 = degp[0].reshape(N, 16)
    d1 = degp[1].reshape(N, 16)
    q1 = _tc_first(d0, d1, x, W1)
    p = _prop_kernel(q1, src_flat, dstp3, dst_flat)
    p0, p1 = p[0].reshape(N, D), p[1].reshape(N, D)
    q2 = _tc_mid(d0, d1, p0, p1, q1, b1.reshape(1, D), W2)
    p = _prop_kernel(q2, src_flat, dstp3, dst_flat)
    p0, p1 = p[0].reshape(N, D), p[1].reshape(N, D)
    q3 = _tc_mid(d0, d1, p0, p1, q2, b2.reshape(1, D), W3)
    p = _prop_kernel(q3, src_flat, dstp3, dst_flat)
    p0, p1 = p[0].reshape(N, D), p[1].reshape(N, D)
    return _tc_final(d0, d1, p0, p1, q3, b3.reshape(1, D), batch2d, Wl,
                     bl.reshape(1, 4))
